# Initial kernel scaffold; baseline (speedup 1.0000x reference)
import jax
import jax.numpy as jnp
from jax.experimental import pallas as pl


def kernel(x, edge_index, W1, b1, W2, b2):
    raise NotImplementedError("write your pallas kernel here")


# speedup vs baseline: 17.4675x; 17.4675x over previous
import functools

import jax
import jax.numpy as jnp
from jax import lax
from jax.experimental import pallas as pl
from jax.experimental.pallas import tpu as pltpu
from jax.experimental.pallas import tpu_sc as plsc

NC = 2
NS = 16
NW = NC * NS
LANES = 16
CHUNK = 128
DEG_W = 16


def _vmesh():
    return plsc.VectorSubcoreMesh(core_axis_name="c", subcore_axis_name="s")


def _fill_zero(ref, rows, width):
    zero = jnp.zeros((LANES,), jnp.float32)

    @pl.loop(0, rows)
    def _(i):
        @pl.loop(0, width, step=LANES)
        def _(j):
            ref[i, pl.ds(j, LANES)] = zero


def _zero_acc(acc_sh, zero_v, n, sid):
    nch = n // CHUNK
    tail = n % CHUNK

    @pl.loop(sid, nch, step=NS)
    def _(c):
        pltpu.sync_copy(zero_v, acc_sh.at[pl.ds(c * CHUNK, CHUNK)])

    if tail:
        @pl.when(sid == 0)
        def _():
            pltpu.sync_copy(zero_v.at[pl.ds(0, tail)],
                            acc_sh.at[pl.ds(nch * CHUNK, tail)])


def _copy_out(acc_sh, out_hbm, n, cid, sid):
    nch = n // CHUNK
    tail = n % CHUNK
    base = cid * n

    @pl.loop(sid, nch, step=NS)
    def _(c):
        pltpu.sync_copy(acc_sh.at[pl.ds(c * CHUNK, CHUNK)],
                        out_hbm.at[pl.ds(base + c * CHUNK, CHUNK)])

    if tail:
        @pl.when(sid == 0)
        def _():
            pltpu.sync_copy(acc_sh.at[pl.ds(nch * CHUNK, tail)],
                            out_hbm.at[pl.ds(base + nch * CHUNK, tail)])


def _sc_degree(dst, n):
    e = dst.shape[0]
    assert e % CHUNK == 0
    nchunks = e // CHUNK

    @functools.partial(
        pl.kernel,
        out_type=jax.ShapeDtypeStruct((NC * n, DEG_W), jnp.float32),
        mesh=_vmesh(),
        scratch_types=[
            pltpu.VMEM((CHUNK,), jnp.int32),
            pltpu.VMEM((CHUNK, DEG_W), jnp.float32),
            pltpu.VMEM((CHUNK, DEG_W), jnp.float32),
            pltpu.VMEM_SHARED((n, DEG_W), jnp.float32),
        ],
    )
    def deg_kernel(dst_hbm, out_hbm, idx_v, ones_v, zero_v, acc_sh):
        cid = lax.axis_index("c")
        sid = lax.axis_index("s")
        wid = sid * NC + cid

        one = jnp.full((LANES,), 1.0, jnp.float32)

        @pl.loop(0, CHUNK)
        def _(i):
            ones_v[i, :] = one

        _fill_zero(zero_v, CHUNK, DEG_W)
        _zero_acc(acc_sh, zero_v, n, sid)
        plsc.subcore_barrier()

        @pl.loop(wid, nchunks, step=NW)
        def _(c):
            pltpu.sync_copy(dst_hbm.at[pl.ds(c * CHUNK, CHUNK)], idx_v)
            pltpu.sync_copy(ones_v, acc_sh.at[idx_v], add=True)

        plsc.subcore_barrier()
        _copy_out(acc_sh, out_hbm, n, cid, sid)

    return deg_kernel(dst)


def _sc_edge_scatter(g, src, dst):
    n, d = g.shape
    e = src.shape[0]
    assert e % CHUNK == 0
    nchunks = e // CHUNK

    @functools.partial(
        pl.kernel,
        out_type=jax.ShapeDtypeStruct((NC * n, d), jnp.float32),
        mesh=_vmesh(),
        compiler_params=pltpu.CompilerParams(use_tc_tiling_on_sc=False),
        scratch_types=[
            pltpu.VMEM((CHUNK,), jnp.int32),
            pltpu.VMEM((CHUNK,), jnp.int32),
            pltpu.VMEM((CHUNK, d), jnp.float32),
            pltpu.VMEM((CHUNK, d), jnp.float32),
            pltpu.VMEM_SHARED((n, d), jnp.float32),
            pltpu.SemaphoreType.DMA,
        ],
    )
    def edge_kernel(g_hbm, src_hbm, dst_hbm, out_hbm,
                    sidx_v, didx_v, rows_v, zero_v, acc_sh, sem):
        cid = lax.axis_index("c")
        sid = lax.axis_index("s")
        wid = sid * NC + cid

        _fill_zero(zero_v, CHUNK, d)
        _zero_acc(acc_sh, zero_v, n, sid)
        plsc.subcore_barrier()

        @pl.loop(wid, nchunks, step=NW)
        def _(c):
            off = c * CHUNK
            pltpu.sync_copy(src_hbm.at[pl.ds(off, CHUNK)], sidx_v)
            pltpu.sync_copy(dst_hbm.at[pl.ds(off, CHUNK)], didx_v)
            pltpu.async_copy(g_hbm.at[sidx_v], rows_v, sem).wait()
            pltpu.sync_copy(rows_v, acc_sh.at[didx_v], add=True)

        plsc.subcore_barrier()
        _copy_out(acc_sh, out_hbm, n, cid, sid)

    return edge_kernel(g, src, dst)


def _tc_matmul(x, w):
    n = x.shape[0]
    dout = w.shape[1]

    def body(x_ref, w_ref, o_ref):
        o_ref[...] = jnp.dot(x_ref[...], w_ref[...],
                             preferred_element_type=jnp.float32)

    return pl.pallas_call(
        body,
        out_shape=jax.ShapeDtypeStruct((n, dout), jnp.float32),
    )(x, w)


def _tc_scale(h, degp):
    n, d = h.shape

    def body(h_ref, degp_ref, g_ref, dinv_ref):
        deg = 1.0 + degp_ref[:n, 0] + degp_ref[n:, 0]
        dinv = lax.rsqrt(deg)
        dinv_ref[...] = dinv[:, None]
        g_ref[...] = h_ref[...] * dinv[:, None]

    return pl.pallas_call(
        body,
        out_shape=(
            jax.ShapeDtypeStruct((n, d), jnp.float32),
            jax.ShapeDtypeStruct((n, 1), jnp.float32),
        ),
    )(h, degp)


def _tc_mid(s1, g1, dinv, b1, w2):
    n, d = g1.shape
    dout = w2.shape[1]

    def body(s_ref, g_ref, dinv_ref, b_ref, w_ref, o_ref):
        acc = s_ref[:n, :] + s_ref[n:, :] + g_ref[...]
        z = jnp.maximum(acc * dinv_ref[...] + b_ref[...][None, :], 0.0)
        o_ref[...] = jnp.dot(z, w_ref[...],
                             preferred_element_type=jnp.float32) * dinv_ref[...]

    return pl.pallas_call(
        body,
        out_shape=jax.ShapeDtypeStruct((n, dout), jnp.float32),
    )(s1, g1, dinv, b1, w2)


def _tc_final(s2, g2, dinv, b2):
    n, d = g2.shape

    def body(s_ref, g_ref, dinv_ref, b_ref, o_ref):
        acc = s_ref[:n, :] + s_ref[n:, :] + g_ref[...]
        o_ref[...] = jnp.maximum(acc * dinv_ref[...] + b_ref[...][None, :], 0.0)

    return pl.pallas_call(
        body,
        out_shape=jax.ShapeDtypeStruct((n, d), jnp.float32),
    )(s2, g2, dinv, b2)


@jax.jit
def kernel(x, edge_index, W1, b1, W2, b2):
    n = x.shape[0]
    src = edge_index[0]
    dst = edge_index[1]

    degp = _sc_degree(dst, n)
    h1 = _tc_matmul(x, W1)
    g1, dinv = _tc_scale(h1, degp)
    s1 = _sc_edge_scatter(g1, src, dst)
    g2 = _tc_mid(s1, g1, dinv, b1, W2)
    s2 = _sc_edge_scatter(g2, src, dst)
    return _tc_final(s2, g2, dinv, b2)
